# baseline (speedup 1.0000x reference)
import functools

import jax
import jax.numpy as jnp
from jax import lax
from jax.experimental import pallas as pl
from jax.experimental.pallas import tpu as pltpu
from jax.experimental.pallas import tpu_sc as plsc

_F = 26
_V = 4000
_W = 4096
_D = 16
_B = 4096
_P = _F * (_F - 1) // 2

_NC = 2
_NS = 16
_NW = _NC * _NS

_DH = 8
_NT = _P * 2
_TPW = _NT // _NW
_XTRA = _NT - _TPW * _NW

_mesh = plsc.VectorSubcoreMesh(core_axis_name="c", subcore_axis_name="s")


@functools.partial(
    pl.kernel,
    mesh=_mesh,
    out_type=jax.ShapeDtypeStruct((_P, _D, _B), jnp.float32),
    compiler_params=pltpu.CompilerParams(
        use_tc_tiling_on_sc=True, needs_layout_passes=False),
    scratch_types=[
        pltpu.VMEM((_DH, _W), jnp.float32),
        pltpu.VMEM((_DH, _W), jnp.float32),
        pltpu.VMEM((_DH, _B), jnp.float32),
        pltpu.VMEM((2, 1, _B), jnp.int32),
        pltpu.VMEM((2, 1, _B), jnp.int32),
        pltpu.SemaphoreType.DMA,
        pltpu.SemaphoreType.DMA,
        pltpu.SemaphoreType.DMA,
    ],
)
def _ffm_sc(wt_hbm, xc_hbm, out_hbm, sa_v, sb_v, out_v, xa_v, xb_v,
            ld, xld, st):
    wid = lax.axis_index("s") * _NC + lax.axis_index("c")
    start = wid * _TPW + jnp.minimum(wid, _XTRA)
    cnt = jnp.where(wid < _XTRA, _TPW + 1, _TPW)

    def unpack(t):
        p = t // 2
        h = t - p * 2

        def bi(_, c):
            i0, rem = c
            n = (_F - 1) - i0
            take = rem >= n
            return (jnp.where(take, i0 + 1, i0), jnp.where(take, rem - n, rem))

        i0, rem = lax.fori_loop(0, _F, bi, (jnp.int32(0), p))
        return (i0, i0 + 1 + rem, p, h)

    def adv(s):
        i, j, p, h = s
        h1 = h + 1
        hw = h1 == 2
        h1 = jnp.where(hw, 0, h1)
        p1 = jnp.where(hw, p + 1, p)
        j1 = jnp.where(hw, j + 1, j)
        jw = hw & (j1 == _F)
        i1 = jnp.where(jw, i + 1, i)
        j1 = jnp.where(jw, i1 + 1, j1)
        return (i1, j1, p1, h1)

    def col0(i):
        return pl.multiple_of((i * _V) & ~jnp.int32(127), 128)

    def slab_descs(s):
        i, j, _, h = s
        d0 = pl.multiple_of(h * _DH, 8)
        hw = _W // 2
        descs = []
        for k in range(2):
            descs.append(pltpu.make_async_copy(
                wt_hbm.at[j, pl.ds(d0, _DH), pl.ds(col0(i) + k * hw, hw)],
                sa_v.at[:, pl.ds(k * hw, hw)], ld))
            descs.append(pltpu.make_async_copy(
                wt_hbm.at[i, pl.ds(d0, _DH), pl.ds(col0(j) + k * hw, hw)],
                sb_v.at[:, pl.ds(k * hw, hw)], ld))
        return tuple(descs)

    def x_descs(s, tp):
        i, j, _, _ = s
        return (
            pltpu.make_async_copy(xc_hbm.at[i], xa_v.at[tp], xld),
            pltpu.make_async_copy(xc_hbm.at[j], xb_v.at[tp], xld),
        )

    def compute(s, tp):
        i, j, _, _ = s
        da = i * _V - col0(i)
        db = j * _V - col0(j)
        rows = [jnp.full((16,), d, jnp.int32) for d in range(_DH)]

        @plsc.parallel_loop(0, _B // 16, unroll=4)
        def block_body(b0):
            xa = xa_v[tp, 0, pl.ds(b0 * 16, 16)] + da
            xb = xb_v[tp, 0, pl.ds(b0 * 16, 16)] + db
            for d in range(_DH):
                va = plsc.load_gather(sa_v, [rows[d], xa])
                vb = plsc.load_gather(sb_v, [rows[d], xb])
                out_v[d, pl.ds(b0 * 16, 16)] = va * vb

    def store_desc(s):
        _, _, p, h = s
        return pltpu.make_async_copy(
            out_v, out_hbm.at[p, pl.ds(pl.multiple_of(h * _DH, 8), _DH)], st)

    s0 = unpack(start)
    for dsc in x_descs(s0, lax.rem(start, 2)):
        dsc.start()

    def step(m, s):
        t = start + m
        tp = lax.rem(t, 2)
        s_nxt = adv(s)

        for dsc in slab_descs(s):
            dsc.start()
        for dsc in slab_descs(s) + x_descs(s, tp):
            dsc.wait()

        @pl.when(m + 1 < cnt)
        def _():
            for dsc in x_descs(s_nxt, lax.rem(t + 1, 2)):
                dsc.start()

        @pl.when(m > 0)
        def _():
            store_desc(s).wait()

        compute(s, tp)
        store_desc(s).start()
        return s_nxt

    lax.fori_loop(0, cnt, step, s0)
    pltpu.make_async_copy(out_v, out_hbm.at[0, pl.ds(0, _DH)], st).wait()


def kernel(x, W):
    wt = W.transpose(0, 2, 1)
    xc = x.T.reshape(_F, 1, _B)
    out_t = _ffm_sc(wt, xc)
    return out_t.transpose(2, 0, 1)


# speedup vs baseline: 1.0031x; 1.0031x over previous
import functools

import jax
import jax.numpy as jnp
from jax import lax
from jax.experimental import pallas as pl
from jax.experimental.pallas import tpu as pltpu
from jax.experimental.pallas import tpu_sc as plsc

_F = 26
_V = 4000
_W = 4096
_D = 16
_B = 4096
_P = _F * (_F - 1) // 2

_NC = 2
_NS = 16
_NW = _NC * _NS

_DH = 8
_NT = _P * 2
_TPW = _NT // _NW
_XTRA = _NT - _TPW * _NW

_mesh = plsc.VectorSubcoreMesh(core_axis_name="c", subcore_axis_name="s")


@functools.partial(
    pl.kernel,
    mesh=_mesh,
    out_type=jax.ShapeDtypeStruct((_P, _D, _B), jnp.float32),
    compiler_params=pltpu.CompilerParams(
        use_tc_tiling_on_sc=True, needs_layout_passes=False),
    scratch_types=[
        pltpu.VMEM((_DH, _W), jnp.float32),
        pltpu.VMEM((_DH, _W), jnp.float32),
        pltpu.VMEM((_DH, _B), jnp.float32),
        pltpu.VMEM((2, 1, _B), jnp.int32),
        pltpu.VMEM((2, 1, _B), jnp.int32),
        pltpu.SemaphoreType.DMA,
        pltpu.SemaphoreType.DMA,
        pltpu.SemaphoreType.DMA,
    ],
)
def _ffm_sc(wt_hbm, xc_hbm, out_hbm, sa_v, sb_v, out_v, xa_v, xb_v,
            ld, xld, st):
    wid = lax.axis_index("s") * _NC + lax.axis_index("c")
    start = wid * _TPW + jnp.minimum(wid, _XTRA)
    cnt = jnp.where(wid < _XTRA, _TPW + 1, _TPW)

    def unpack(t):
        p = t // 2
        h = t - p * 2

        def bi(_, c):
            i0, rem = c
            n = (_F - 1) - i0
            take = rem >= n
            return (jnp.where(take, i0 + 1, i0), jnp.where(take, rem - n, rem))

        i0, rem = lax.fori_loop(0, _F, bi, (jnp.int32(0), p))
        return (i0, i0 + 1 + rem, p, h)

    def adv(s):
        i, j, p, h = s
        h1 = h + 1
        hw = h1 == 2
        h1 = jnp.where(hw, 0, h1)
        p1 = jnp.where(hw, p + 1, p)
        j1 = jnp.where(hw, j + 1, j)
        jw = hw & (j1 == _F)
        i1 = jnp.where(jw, i + 1, i)
        j1 = jnp.where(jw, i1 + 1, j1)
        return (i1, j1, p1, h1)

    def col0(i):
        return pl.multiple_of((i * _V) & ~jnp.int32(127), 128)

    def slab_descs(s):
        i, j, _, h = s
        d0 = pl.multiple_of(h * _DH, 8)
        return (
            pltpu.make_async_copy(
                wt_hbm.at[j, pl.ds(d0, _DH), pl.ds(col0(i), _W)],
                sa_v, ld),
            pltpu.make_async_copy(
                wt_hbm.at[i, pl.ds(d0, _DH), pl.ds(col0(j), _W)],
                sb_v, ld),
        )

    def x_descs(s, tp):
        i, j, _, _ = s
        return (
            pltpu.make_async_copy(xc_hbm.at[i], xa_v.at[tp], xld),
            pltpu.make_async_copy(xc_hbm.at[j], xb_v.at[tp], xld),
        )

    def compute(s, tp):
        i, j, _, _ = s
        da = i * _V - col0(i)
        db = j * _V - col0(j)
        rows = [jnp.full((16,), d, jnp.int32) for d in range(_DH)]

        @plsc.parallel_loop(0, _B // 16, unroll=4)
        def block_body(b0):
            xa = xa_v[tp, 0, pl.ds(b0 * 16, 16)] + da
            xb = xb_v[tp, 0, pl.ds(b0 * 16, 16)] + db
            for d in range(_DH):
                va = plsc.load_gather(sa_v, [rows[d], xa])
                vb = plsc.load_gather(sb_v, [rows[d], xb])
                out_v[d, pl.ds(b0 * 16, 16)] = va * vb

    def store_desc(s):
        _, _, p, h = s
        return pltpu.make_async_copy(
            out_v, out_hbm.at[p, pl.ds(pl.multiple_of(h * _DH, 8), _DH)], st)

    s0 = unpack(start)
    for dsc in x_descs(s0, lax.rem(start, 2)):
        dsc.start()

    def step(m, s):
        t = start + m
        tp = lax.rem(t, 2)
        s_nxt = adv(s)

        for dsc in slab_descs(s):
            dsc.start()
        for dsc in slab_descs(s) + x_descs(s, tp):
            dsc.wait()

        @pl.when(m + 1 < cnt)
        def _():
            for dsc in x_descs(s_nxt, lax.rem(t + 1, 2)):
                dsc.start()

        @pl.when(m > 0)
        def _():
            store_desc(s).wait()

        compute(s, tp)
        store_desc(s).start()
        return s_nxt

    lax.fori_loop(0, cnt, step, s0)
    pltpu.make_async_copy(out_v, out_hbm.at[0, pl.ds(0, _DH)], st).wait()


def kernel(x, W):
    wt = W.transpose(0, 2, 1)
    xc = x.T.reshape(_F, 1, _B)
    out_t = _ffm_sc(wt, xc)
    return out_t.transpose(2, 0, 1)


# speedup vs baseline: 1.2922x; 1.2882x over previous
import functools

import jax
import jax.numpy as jnp
from jax import lax
from jax.experimental import pallas as pl
from jax.experimental.pallas import tpu as pltpu
from jax.experimental.pallas import tpu_sc as plsc

_F = 26
_V = 4000
_W = 4096
_D = 16
_B = 4096
_P = _F * (_F - 1) // 2

_NC = 2
_NS = 16
_NW = _NC * _NS

_DH = 8
_NT = _P * 2
_TPW = _NT // _NW
_XTRA = _NT - _TPW * _NW

_mesh = plsc.VectorSubcoreMesh(core_axis_name="c", subcore_axis_name="s")


@functools.partial(
    pl.kernel,
    mesh=_mesh,
    out_type=jax.ShapeDtypeStruct((_P, _D, _B), jnp.float32),
    compiler_params=pltpu.CompilerParams(
        use_tc_tiling_on_sc=True, needs_layout_passes=False),
    scratch_types=[
        pltpu.VMEM((_DH, _W), jnp.float32),
        pltpu.VMEM((_DH, _W), jnp.float32),
        pltpu.VMEM((_DH, _B), jnp.float32),
        pltpu.VMEM((2, 1, _B), jnp.int32),
        pltpu.VMEM((2, 1, _B), jnp.int32),
        pltpu.SemaphoreType.DMA,
        pltpu.SemaphoreType.DMA,
        pltpu.SemaphoreType.DMA,
        pltpu.SemaphoreType.DMA,
    ],
)
def _ffm_sc(wt_hbm, xc_hbm, out_hbm, sa_v, sb_v, out_v, xa_v, xb_v,
            lda, ldb, xld, st):
    wid = lax.axis_index("s") * _NC + lax.axis_index("c")
    start = wid * _TPW + jnp.minimum(wid, _XTRA)
    cnt = jnp.where(wid < _XTRA, _TPW + 1, _TPW)

    def unpack(t):
        p = t // 2
        h = t - p * 2

        def bi(_, c):
            i0, rem = c
            n = (_F - 1) - i0
            take = rem >= n
            return (jnp.where(take, i0 + 1, i0), jnp.where(take, rem - n, rem))

        i0, rem = lax.fori_loop(0, _F, bi, (jnp.int32(0), p))
        return (i0, i0 + 1 + rem, p, h)

    def adv(s):
        i, j, p, h = s
        h1 = h + 1
        hw = h1 == 2
        h1 = jnp.where(hw, 0, h1)
        p1 = jnp.where(hw, p + 1, p)
        j1 = jnp.where(hw, j + 1, j)
        jw = hw & (j1 == _F)
        i1 = jnp.where(jw, i + 1, i)
        j1 = jnp.where(jw, i1 + 1, j1)
        return (i1, j1, p1, h1)

    def col0(i):
        return pl.multiple_of((i * _V) & ~jnp.int32(127), 128)

    def slab_a_desc(s):
        i, j, _, h = s
        d0 = pl.multiple_of(h * _DH, 8)
        return pltpu.make_async_copy(
            wt_hbm.at[j, pl.ds(d0, _DH), pl.ds(col0(i), _W)], sa_v, lda)

    def slab_b_desc(s):
        i, j, _, h = s
        d0 = pl.multiple_of(h * _DH, 8)
        return pltpu.make_async_copy(
            wt_hbm.at[i, pl.ds(d0, _DH), pl.ds(col0(j), _W)], sb_v, ldb)

    def x_descs(s, tp):
        i, j, _, _ = s
        return (
            pltpu.make_async_copy(xc_hbm.at[i], xa_v.at[tp], xld),
            pltpu.make_async_copy(xc_hbm.at[j], xb_v.at[tp], xld),
        )

    def compute_a(s, tp):
        i, _, _, _ = s
        da = i * _V - col0(i)
        rows = [jnp.full((16,), d, jnp.int32) for d in range(_DH)]

        @plsc.parallel_loop(0, _B // 16, unroll=2)
        def block_body(b0):
            xa = xa_v[tp, 0, pl.ds(b0 * 16, 16)] + da
            for d in range(_DH):
                out_v[d, pl.ds(b0 * 16, 16)] = plsc.load_gather(
                    sa_v, [rows[d], xa])

    def compute_b(s, tp):
        _, j, _, _ = s
        db = j * _V - col0(j)
        rows = [jnp.full((16,), d, jnp.int32) for d in range(_DH)]

        @plsc.parallel_loop(0, _B // 16, unroll=2)
        def block_body(b0):
            xb = xb_v[tp, 0, pl.ds(b0 * 16, 16)] + db
            for d in range(_DH):
                blk = pl.ds(b0 * 16, 16)
                out_v[d, blk] = out_v[d, blk] * plsc.load_gather(
                    sb_v, [rows[d], xb])

    def store_desc(s):
        _, _, p, h = s
        return pltpu.make_async_copy(
            out_v, out_hbm.at[p, pl.ds(pl.multiple_of(h * _DH, 8), _DH)], st)

    s0 = unpack(start)
    slab_a_desc(s0).start()
    slab_b_desc(s0).start()
    for dsc in x_descs(s0, lax.rem(start, 2)):
        dsc.start()

    def step(m, s):
        t = start + m
        tp = lax.rem(t, 2)
        s_nxt = adv(s)

        slab_a_desc(s).wait()
        for dsc in x_descs(s, tp):
            dsc.wait()

        @pl.when(m > 0)
        def _():
            store_desc(s).wait()

        compute_a(s, tp)

        @pl.when(m + 1 < cnt)
        def _():
            slab_a_desc(s_nxt).start()
            for dsc in x_descs(s_nxt, lax.rem(t + 1, 2)):
                dsc.start()

        slab_b_desc(s).wait()
        compute_b(s, tp)
        store_desc(s).start()

        @pl.when(m + 1 < cnt)
        def _():
            slab_b_desc(s_nxt).start()

        return s_nxt

    lax.fori_loop(0, cnt, step, s0)
    pltpu.make_async_copy(out_v, out_hbm.at[0, pl.ds(0, _DH)], st).wait()


def kernel(x, W):
    wt = W.transpose(0, 2, 1)
    xc = x.T.reshape(_F, 1, _B)
    out_t = _ffm_sc(wt, xc)
    return out_t.transpose(2, 0, 1)
